# Initial kernel scaffold; baseline (speedup 1.0000x reference)
import jax
import jax.numpy as jnp
from jax.experimental import pallas as pl


def kernel(inp, embedding, w_ih, w_hh, b_gate, w_map, b_map, w_head, b_head):
    raise NotImplementedError("write your pallas kernel here")


# speedup vs baseline: 3.4589x; 3.4589x over previous
import functools

import jax
import jax.numpy as jnp
from jax.experimental import pallas as pl
from jax.experimental.pallas import tpu as pltpu


def _round_up(n, m):
    return ((n + m - 1) // m) * m


def _rmtpp_fused_kernel(time_ref, ev_ref, embT_ref, w_ihT_ref, w_hhT_ref,
                        bT_ref, w_map_ref, b_map_ref, w_head_ref, b_head_ref,
                        out_ref, *, hidden, num_classes):
    f32 = jnp.float32
    S, Bt = time_ref.shape
    H = hidden
    C = num_classes

    embT = embT_ref[...]
    w_ihT = w_ihT_ref[...]
    w_hhT = w_hhT_ref[...]
    bT = bT_ref[...]
    mm = w_hhT.dtype
    Dp = embT.shape[0]

    hT = jnp.zeros((H, Bt), f32)
    cT = jnp.zeros((H, Bt), f32)

    row0 = jax.lax.broadcasted_iota(jnp.int32, (Dp, Bt), 0) == 0
    iota_c = jax.lax.broadcasted_iota(jnp.int32, (C, Bt), 0)

    for t in range(S):
        ev = ev_ref[t:t + 1, :]
        tm = time_ref[t:t + 1, :]
        onehot = (iota_c == ev).astype(mm)
        xT = jnp.dot(embT, onehot, preferred_element_type=f32)
        xT = jnp.where(row0, tm, xT).astype(mm)
        gT = (jnp.dot(w_ihT, xT, preferred_element_type=f32)
              + jnp.dot(w_hhT, hT.astype(mm), preferred_element_type=f32)
              + bT)
        sig = jax.nn.sigmoid(gT[:3 * H])
        i_g = sig[0 * H:1 * H]
        f_g = sig[1 * H:2 * H]
        o_g = sig[2 * H:3 * H]
        g_g = jnp.tanh(gT[3 * H:4 * H])
        cT = f_g * cT + i_g * g_g
        hT = o_g * jnp.tanh(cT)

    h = hT.T.astype(mm)
    map_out = jax.nn.sigmoid(
        jnp.dot(h, w_map_ref[...], preferred_element_type=f32)
        + b_map_ref[...])
    head = (jnp.dot(map_out.astype(mm), w_head_ref[...],
                    preferred_element_type=f32)
            + b_head_ref[...])

    col = jax.lax.broadcasted_iota(jnp.int32, head.shape, 1)
    evt_mask = col < C
    masked = jnp.where(evt_mask, head, jnp.float32(-1e30))
    m = jnp.max(masked, axis=1, keepdims=True)
    p = jnp.where(evt_mask, jnp.exp(masked - m), 0.0)
    lse = m + jnp.log(jnp.sum(p, axis=1, keepdims=True))
    out_ref[...] = jnp.where(evt_mask, head - lse, head)


def kernel(inp, embedding, w_ih, w_hh, b_gate, w_map, b_map, w_head, b_head):
    C, E = embedding.shape
    H = w_hh.shape[0]
    Dp, Gp = w_ih.shape
    Mp = w_map.shape[1]
    HEAD = w_head.shape[1]
    Din = E + 1
    mm = w_ih.dtype
    B, S, _ = inp.shape

    time_seq = inp[:, :, 0]
    ev_seq = inp[:, :, 1].astype(jnp.int32)

    Bt = 512
    B_pad = _round_up(B, Bt)

    time_sb = jnp.pad(time_seq.T, ((0, 0), (0, B_pad - B)))
    ev_sb = jnp.pad(ev_seq.T, ((0, 0), (0, B_pad - B)))

    embT = jnp.pad(embedding.astype(mm).T, ((1, Dp - Din), (0, 0)))

    kfn = functools.partial(_rmtpp_fused_kernel, hidden=H, num_classes=C)
    const2 = lambda i: (0, 0)
    out = pl.pallas_call(
        kfn,
        out_shape=jax.ShapeDtypeStruct((B_pad, HEAD), jnp.float32),
        grid=(B_pad // Bt,),
        in_specs=[
            pl.BlockSpec((S, Bt), lambda i: (0, i)),
            pl.BlockSpec((S, Bt), lambda i: (0, i)),
            pl.BlockSpec((Dp, C), const2),
            pl.BlockSpec((Gp, Dp), const2),
            pl.BlockSpec((Gp, H), const2),
            pl.BlockSpec((Gp, 1), const2),
            pl.BlockSpec((H, Mp), const2),
            pl.BlockSpec((1, Mp), const2),
            pl.BlockSpec((Mp, HEAD), const2),
            pl.BlockSpec((1, HEAD), const2),
        ],
        out_specs=pl.BlockSpec((Bt, HEAD), lambda i: (i, 0)),
        compiler_params=pltpu.CompilerParams(
            dimension_semantics=("parallel",),
            vmem_limit_bytes=48 << 20),
    )(time_sb, ev_sb, embT, w_ih.T, w_hh.T, b_gate.T,
      w_map, b_map, w_head, b_head)

    event_out = out[:B, :C]
    time_out = out[:B, C:C + 1]
    last_time = inp[:, -1, 0:1]
    return time_out, event_out, last_time


# speedup vs baseline: 3.9428x; 1.1399x over previous
import functools

import jax
import jax.numpy as jnp
from jax.experimental import pallas as pl
from jax.experimental.pallas import tpu as pltpu


def _round_up(n, m):
    return ((n + m - 1) // m) * m


def _rmtpp_fused_kernel(time_ref, ev_ref, embT_ref, w_ihT_ref, w_hhT_ref,
                        bT_ref, w_map_ref, b_map_ref, w_head_ref, b_head_ref,
                        out_ref, *, hidden, num_classes):
    f32 = jnp.float32
    S, Bt = time_ref.shape
    H = hidden
    C = num_classes

    embT = embT_ref[...]
    w_ihT = w_ihT_ref[...]
    w_hhT = w_hhT_ref[...]
    bT = bT_ref[...]
    mm = w_hhT.dtype
    Dp = embT.shape[0]

    NC = 2
    Bh = Bt // NC
    row0 = jax.lax.broadcasted_iota(jnp.int32, (Dp, Bh), 0) == 0
    iota_c = jax.lax.broadcasted_iota(jnp.int32, (C, Bh), 0)

    def step(ev, tm, h, c):
        onehot = (iota_c == ev).astype(mm)
        xT = jnp.dot(embT, onehot, preferred_element_type=f32)
        xT = jnp.where(row0, tm, xT).astype(mm)
        gT = (jnp.dot(w_ihT, xT, preferred_element_type=f32)
              + jnp.dot(w_hhT, h.astype(mm), preferred_element_type=f32)
              + bT)
        sig = 0.5 + 0.5 * jnp.tanh(0.5 * gT[:3 * H])
        i_g = sig[0 * H:1 * H]
        f_g = sig[1 * H:2 * H]
        o_g = sig[2 * H:3 * H]
        g_g = jnp.tanh(gT[3 * H:4 * H])
        c = f_g * c + i_g * g_g
        h = o_g * jnp.tanh(c)
        return h, c

    hs = [jnp.zeros((H, Bh), f32)] * NC
    cs = [jnp.zeros((H, Bh), f32)] * NC
    for t in range(S):
        for k in range(NC):
            hs[k], cs[k] = step(ev_ref[t:t + 1, k * Bh:(k + 1) * Bh],
                                time_ref[t:t + 1, k * Bh:(k + 1) * Bh],
                                hs[k], cs[k])

    hT = jnp.concatenate(hs, axis=1)
    h = hT.T.astype(mm)
    map_out = jax.nn.sigmoid(
        jnp.dot(h, w_map_ref[...], preferred_element_type=f32)
        + b_map_ref[...])
    head = (jnp.dot(map_out.astype(mm), w_head_ref[...],
                    preferred_element_type=f32)
            + b_head_ref[...])

    col = jax.lax.broadcasted_iota(jnp.int32, head.shape, 1)
    evt_mask = col < C
    masked = jnp.where(evt_mask, head, jnp.float32(-1e30))
    m = jnp.max(masked, axis=1, keepdims=True)
    p = jnp.where(evt_mask, jnp.exp(masked - m), 0.0)
    lse = m + jnp.log(jnp.sum(p, axis=1, keepdims=True))
    out_ref[...] = jnp.where(evt_mask, head - lse, head)


def kernel(inp, embedding, w_ih, w_hh, b_gate, w_map, b_map, w_head, b_head):
    C, E = embedding.shape
    H = w_hh.shape[0]
    Dp, Gp = w_ih.shape
    Mp = w_map.shape[1]
    HEAD = w_head.shape[1]
    Din = E + 1
    mm = w_ih.dtype
    B, S, _ = inp.shape

    time_seq = inp[:, :, 0]
    ev_seq = inp[:, :, 1].astype(jnp.int32)

    Bt = 512
    B_pad = _round_up(B, Bt)

    time_sb = jnp.pad(time_seq.T, ((0, 0), (0, B_pad - B)))
    ev_sb = jnp.pad(ev_seq.T, ((0, 0), (0, B_pad - B)))

    embT = jnp.pad(embedding.astype(mm).T, ((1, Dp - Din), (0, 0)))

    kfn = functools.partial(_rmtpp_fused_kernel, hidden=H, num_classes=C)
    const2 = lambda i: (0, 0)
    out = pl.pallas_call(
        kfn,
        out_shape=jax.ShapeDtypeStruct((B_pad, HEAD), jnp.float32),
        grid=(B_pad // Bt,),
        in_specs=[
            pl.BlockSpec((S, Bt), lambda i: (0, i)),
            pl.BlockSpec((S, Bt), lambda i: (0, i)),
            pl.BlockSpec((Dp, C), const2),
            pl.BlockSpec((Gp, Dp), const2),
            pl.BlockSpec((Gp, H), const2),
            pl.BlockSpec((Gp, 1), const2),
            pl.BlockSpec((H, Mp), const2),
            pl.BlockSpec((1, Mp), const2),
            pl.BlockSpec((Mp, HEAD), const2),
            pl.BlockSpec((1, HEAD), const2),
        ],
        out_specs=pl.BlockSpec((Bt, HEAD), lambda i: (i, 0)),
        compiler_params=pltpu.CompilerParams(
            dimension_semantics=("parallel",),
            vmem_limit_bytes=48 << 20),
    )(time_sb, ev_sb, embT, w_ih.T, w_hh.T, b_gate.T,
      w_map, b_map, w_head, b_head)

    event_out = out[:B, :C]
    time_out = out[:B, C:C + 1]
    last_time = inp[:, -1, 0:1]
    return time_out, event_out, last_time


# speedup vs baseline: 4.0323x; 1.0227x over previous
import functools

import jax
import jax.numpy as jnp
from jax.experimental import pallas as pl
from jax.experimental.pallas import tpu as pltpu


def _round_up(n, m):
    return ((n + m - 1) // m) * m

_T0 = 16


def _rmtpp_fused_kernel(time_ref, ev_ref, embT2_ref, w_cat_ref,
                        w_map_ref, b_map_ref, w_head_ref, b_head_ref,
                        out_ref, *, hidden, num_classes):
    f32 = jnp.float32
    S, Bt = time_ref.shape
    H = hidden
    C = num_classes

    embT2 = embT2_ref[...]
    w_cat = w_cat_ref[...]
    mm = w_cat.dtype

    NC = 2
    Bh = Bt // NC
    iota_c = jax.lax.broadcasted_iota(jnp.int32, (C, Bh), 0)
    iota_t = jax.lax.broadcasted_iota(jnp.int32, (_T0, Bh), 0)

    def step(ev, tm, h, c):
        onehot = (iota_c == ev).astype(mm)
        x2 = jnp.dot(embT2, onehot, preferred_element_type=f32).astype(mm)
        tmrow = jnp.where(iota_t == 0, tm, 0.0).astype(mm)
        xh = jnp.concatenate([tmrow, x2, h.astype(mm)], axis=0)
        g = jnp.dot(w_cat, xh, preferred_element_type=f32)
        t3 = jnp.tanh(g[:3 * H])
        i_g = 0.5 * t3[0 * H:1 * H] + 0.5
        f_g = 0.5 * t3[1 * H:2 * H] + 0.5
        o_g = 0.5 * t3[2 * H:3 * H] + 0.5
        g_g = jnp.tanh(g[3 * H:4 * H])
        c = f_g * c + i_g * g_g
        h = o_g * jnp.tanh(c)
        return h, c

    hs = [jnp.zeros((H, Bh), f32)] * NC
    cs = [jnp.zeros((H, Bh), f32)] * NC
    for t in range(S):
        for k in range(NC):
            hs[k], cs[k] = step(ev_ref[t:t + 1, k * Bh:(k + 1) * Bh],
                                time_ref[t:t + 1, k * Bh:(k + 1) * Bh],
                                hs[k], cs[k])

    hT = jnp.concatenate(hs, axis=1)
    h = hT.T.astype(mm)
    map_out = jax.nn.sigmoid(
        jnp.dot(h, w_map_ref[...], preferred_element_type=f32)
        + b_map_ref[...])
    head = (jnp.dot(map_out.astype(mm), w_head_ref[...],
                    preferred_element_type=f32)
            + b_head_ref[...])

    col = jax.lax.broadcasted_iota(jnp.int32, head.shape, 1)
    evt_mask = col < C
    masked = jnp.where(evt_mask, head, jnp.float32(-1e30))
    m = jnp.max(masked, axis=1, keepdims=True)
    p = jnp.where(evt_mask, jnp.exp(masked - m), 0.0)
    lse = m + jnp.log(jnp.sum(p, axis=1, keepdims=True))
    out_ref[...] = jnp.where(evt_mask, head - lse, head)


def kernel(inp, embedding, w_ih, w_hh, b_gate, w_map, b_map, w_head, b_head):
    f32 = jnp.float32
    C, E = embedding.shape
    H = w_hh.shape[0]
    Dp, Gp = w_ih.shape
    Mp = w_map.shape[1]
    HEAD = w_head.shape[1]
    mm = w_ih.dtype
    B, S, _ = inp.shape

    time_seq = inp[:, :, 0]
    ev_seq = inp[:, :, 1].astype(jnp.int32)

    Bt = 512
    B_pad = _round_up(B, Bt)

    time_sb = jnp.pad(time_seq.T, ((0, 0), (0, B_pad - B)))
    ev_sb = jnp.pad(ev_seq.T, ((0, 0), (0, B_pad - B)))

    M2 = _round_up(E + 2, _T0)
    embT2 = jnp.zeros((M2, C), mm)
    embT2 = embT2.at[:E].set(embedding.astype(mm).T)
    embT2 = embT2.at[E].set(jnp.ones((C,), mm))
    embT2 = embT2.at[E + 1].set(jnp.ones((C,), mm))

    wx = w_ih.T
    bvec = b_gate[0].astype(f32)
    b_hi = bvec.astype(mm)
    b_lo = (bvec - b_hi.astype(f32)).astype(mm)
    w_cat = jnp.zeros((Gp, _T0 + M2 + H), mm)
    w_cat = w_cat.at[:, 0].set(wx[:, 0])
    w_cat = w_cat.at[:, _T0:_T0 + E].set(wx[:, 1:1 + E])
    w_cat = w_cat.at[:, _T0 + E].set(b_hi)
    w_cat = w_cat.at[:, _T0 + E + 1].set(b_lo)
    w_cat = w_cat.at[:, _T0 + M2:].set(w_hh.T)
    gate_scale = jnp.where(jnp.arange(Gp) < 3 * H, 0.5, 1.0).astype(mm)
    w_cat = w_cat * gate_scale[:, None]
    K = _T0 + M2 + H

    kfn = functools.partial(_rmtpp_fused_kernel, hidden=H, num_classes=C)
    const2 = lambda i: (0, 0)
    out = pl.pallas_call(
        kfn,
        out_shape=jax.ShapeDtypeStruct((B_pad, HEAD), jnp.float32),
        grid=(B_pad // Bt,),
        in_specs=[
            pl.BlockSpec((S, Bt), lambda i: (0, i)),
            pl.BlockSpec((S, Bt), lambda i: (0, i)),
            pl.BlockSpec((M2, C), const2),
            pl.BlockSpec((Gp, K), const2),
            pl.BlockSpec((H, Mp), const2),
            pl.BlockSpec((1, Mp), const2),
            pl.BlockSpec((Mp, HEAD), const2),
            pl.BlockSpec((1, HEAD), const2),
        ],
        out_specs=pl.BlockSpec((Bt, HEAD), lambda i: (i, 0)),
        compiler_params=pltpu.CompilerParams(
            dimension_semantics=("parallel",),
            vmem_limit_bytes=48 << 20),
    )(time_sb, ev_sb, embT2, w_cat, w_map, b_map, w_head, b_head)

    event_out = out[:B, :C]
    time_out = out[:B, C:C + 1]
    last_time = inp[:, -1, 0:1]
    return time_out, event_out, last_time


# speedup vs baseline: 4.2673x; 1.0583x over previous
import functools

import jax
import jax.numpy as jnp
from jax.experimental import pallas as pl
from jax.experimental.pallas import tpu as pltpu


def _round_up(n, m):
    return ((n + m - 1) // m) * m

_T0 = 16


def _rmtpp_fused_kernel(time_ref, ev_ref, embT2_ref, w_cat_ref,
                        w_map_ref, b_map_ref, w_head_ref, b_head_ref,
                        out_ref, *, hidden, num_classes):
    f32 = jnp.float32
    S, _, Bt = time_ref.shape
    H = hidden
    C = num_classes

    embT2 = embT2_ref[...]
    w_cat = w_cat_ref[...]
    mm = w_cat.dtype

    NC = 4
    Bh = Bt // NC
    iota_c = jax.lax.broadcasted_iota(jnp.int32, (C, Bh), 0)
    iota_t = jax.lax.broadcasted_iota(jnp.int32, (_T0, Bh), 0)

    def step(ev, tm, h, c):
        onehot = (iota_c == ev).astype(mm)
        x2 = jnp.dot(embT2, onehot, preferred_element_type=f32).astype(mm)
        tmrow = jnp.where(iota_t == 0, tm, 0.0).astype(mm)
        xh = jnp.concatenate([tmrow, x2, h.astype(mm)], axis=0)
        g = jnp.dot(w_cat, xh, preferred_element_type=f32)
        t3 = jnp.tanh(g[:3 * H])
        i_g = 0.5 * t3[0 * H:1 * H] + 0.5
        f_g = 0.5 * t3[1 * H:2 * H] + 0.5
        o_g = 0.5 * t3[2 * H:3 * H] + 0.5
        g_g = jnp.tanh(g[3 * H:4 * H])
        c = f_g * c + i_g * g_g
        h = o_g * jnp.tanh(c)
        return h, c

    hs = [jnp.zeros((H, Bh), f32)] * NC
    cs = [jnp.zeros((H, Bh), f32)] * NC
    for t in range(S):
        ev_t = ev_ref[t]
        tm_t = time_ref[t]
        for k in range(NC):
            hs[k], cs[k] = step(ev_t[:, k * Bh:(k + 1) * Bh],
                                tm_t[:, k * Bh:(k + 1) * Bh],
                                hs[k], cs[k])

    hT = jnp.concatenate(hs, axis=1)
    h = hT.T.astype(mm)
    map_out = jax.nn.sigmoid(
        jnp.dot(h, w_map_ref[...], preferred_element_type=f32)
        + b_map_ref[...])
    head = (jnp.dot(map_out.astype(mm), w_head_ref[...],
                    preferred_element_type=f32)
            + b_head_ref[...])

    col = jax.lax.broadcasted_iota(jnp.int32, head.shape, 1)
    evt_mask = col < C
    masked = jnp.where(evt_mask, head, jnp.float32(-1e30))
    m = jnp.max(masked, axis=1, keepdims=True)
    p = jnp.where(evt_mask, jnp.exp(masked - m), 0.0)
    lse = m + jnp.log(jnp.sum(p, axis=1, keepdims=True))
    out_ref[...] = jnp.where(evt_mask, head - lse, head)


def kernel(inp, embedding, w_ih, w_hh, b_gate, w_map, b_map, w_head, b_head):
    f32 = jnp.float32
    C, E = embedding.shape
    H = w_hh.shape[0]
    Dp, Gp = w_ih.shape
    Mp = w_map.shape[1]
    HEAD = w_head.shape[1]
    mm = w_ih.dtype
    B, S, _ = inp.shape

    time_seq = inp[:, :, 0]
    ev_seq = inp[:, :, 1].astype(jnp.int32)

    Bt = 1024
    B_pad = _round_up(B, Bt)

    time_sb = jnp.pad(time_seq.T, ((0, 0), (0, B_pad - B)))[:, None, :]
    ev_sb = jnp.pad(ev_seq.T, ((0, 0), (0, B_pad - B)))[:, None, :]

    M2 = _round_up(E + 2, _T0)
    embT2 = jnp.zeros((M2, C), mm)
    embT2 = embT2.at[:E].set(embedding.astype(mm).T)
    embT2 = embT2.at[E].set(jnp.ones((C,), mm))
    embT2 = embT2.at[E + 1].set(jnp.ones((C,), mm))

    wx = w_ih.T
    bvec = b_gate[0].astype(f32)
    b_hi = bvec.astype(mm)
    b_lo = (bvec - b_hi.astype(f32)).astype(mm)
    w_cat = jnp.zeros((Gp, _T0 + M2 + H), mm)
    w_cat = w_cat.at[:, 0].set(wx[:, 0])
    w_cat = w_cat.at[:, _T0:_T0 + E].set(wx[:, 1:1 + E])
    w_cat = w_cat.at[:, _T0 + E].set(b_hi)
    w_cat = w_cat.at[:, _T0 + E + 1].set(b_lo)
    w_cat = w_cat.at[:, _T0 + M2:].set(w_hh.T)
    gate_scale = jnp.where(jnp.arange(Gp) < 3 * H, 0.5, 1.0).astype(mm)
    w_cat = w_cat * gate_scale[:, None]
    K = _T0 + M2 + H

    kfn = functools.partial(_rmtpp_fused_kernel, hidden=H, num_classes=C)
    const2 = lambda i: (0, 0)
    out = pl.pallas_call(
        kfn,
        out_shape=jax.ShapeDtypeStruct((B_pad, HEAD), jnp.float32),
        grid=(B_pad // Bt,),
        in_specs=[
            pl.BlockSpec((S, 1, Bt), lambda i: (0, 0, i)),
            pl.BlockSpec((S, 1, Bt), lambda i: (0, 0, i)),
            pl.BlockSpec((M2, C), const2),
            pl.BlockSpec((Gp, K), const2),
            pl.BlockSpec((H, Mp), const2),
            pl.BlockSpec((1, Mp), const2),
            pl.BlockSpec((Mp, HEAD), const2),
            pl.BlockSpec((1, HEAD), const2),
        ],
        out_specs=pl.BlockSpec((Bt, HEAD), lambda i: (i, 0)),
        compiler_params=pltpu.CompilerParams(
            dimension_semantics=("parallel",),
            vmem_limit_bytes=48 << 20),
    )(time_sb, ev_sb, embT2, w_cat, w_map, b_map, w_head, b_head)

    event_out = out[:B, :C]
    time_out = out[:B, C:C + 1]
    last_time = inp[:, -1, 0:1]
    return time_out, event_out, last_time


# speedup vs baseline: 5.0271x; 1.1780x over previous
import functools

import jax
import jax.numpy as jnp
from jax.experimental import pallas as pl
from jax.experimental.pallas import tpu as pltpu


def _round_up(n, m):
    return ((n + m - 1) // m) * m

_T0 = 16


def _rmtpp_fused_kernel(time_ref, ev_ref, embT2_ref, w_cat_ref,
                        w_map_ref, b_map_ref, w_head_ref, b_head_ref,
                        out_ref, *, hidden, num_classes):
    f32 = jnp.float32
    S, _, Bt = time_ref.shape
    H = hidden
    C = num_classes

    embT2 = embT2_ref[...]
    w_cat = w_cat_ref[...]
    mm = w_cat.dtype

    NC = 2
    Bh = Bt // NC
    iota_c = jax.lax.broadcasted_iota(jnp.int32, (C, Bh), 0)
    iota_t = jax.lax.broadcasted_iota(jnp.int32, (_T0, Bh), 0)

    def step(ev, tm, h, c):
        onehot = (iota_c == ev).astype(mm)
        x2 = jnp.dot(embT2, onehot, preferred_element_type=f32).astype(mm)
        tmrow = jnp.where(iota_t == 0, tm, 0.0).astype(mm)
        xh = jnp.concatenate([tmrow, x2, h.astype(mm)], axis=0)
        g = jnp.dot(w_cat, xh, preferred_element_type=f32)
        t3 = jnp.tanh(g[:3 * H])
        i_g = 0.5 * t3[0 * H:1 * H] + 0.5
        f_g = 0.5 * t3[1 * H:2 * H] + 0.5
        o_g = 0.5 * t3[2 * H:3 * H] + 0.5
        g_g = jnp.tanh(g[3 * H:4 * H])
        c = f_g * c + i_g * g_g
        h = o_g * jnp.tanh(c)
        return h, c

    hs = [jnp.zeros((H, Bh), f32)] * NC
    cs = [jnp.zeros((H, Bh), f32)] * NC
    for t in range(S):
        ev_t = ev_ref[t]
        tm_t = time_ref[t]
        for k in range(NC):
            hs[k], cs[k] = step(ev_t[:, k * Bh:(k + 1) * Bh],
                                tm_t[:, k * Bh:(k + 1) * Bh],
                                hs[k], cs[k])

    hT = jnp.concatenate(hs, axis=1)
    h = hT.T.astype(mm)
    map_out = jax.nn.sigmoid(
        jnp.dot(h, w_map_ref[...], preferred_element_type=f32)
        + b_map_ref[...])
    head = (jnp.dot(map_out.astype(mm), w_head_ref[...],
                    preferred_element_type=f32)
            + b_head_ref[...])

    col = jax.lax.broadcasted_iota(jnp.int32, head.shape, 1)
    evt_mask = col < C
    masked = jnp.where(evt_mask, head, jnp.float32(-1e30))
    m = jnp.max(masked, axis=1, keepdims=True)
    p = jnp.where(evt_mask, jnp.exp(masked - m), 0.0)
    lse = m + jnp.log(jnp.sum(p, axis=1, keepdims=True))
    out_ref[...] = jnp.where(evt_mask, head - lse, head)


def kernel(inp, embedding, w_ih, w_hh, b_gate, w_map, b_map, w_head, b_head):
    f32 = jnp.float32
    C, E = embedding.shape
    H = w_hh.shape[0]
    Dp, Gp = w_ih.shape
    Mp = w_map.shape[1]
    HEAD = w_head.shape[1]
    mm = w_ih.dtype
    B, S, _ = inp.shape

    time_seq = inp[:, :, 0]
    ev_seq = inp[:, :, 1].astype(jnp.int32)

    Bt = 2048
    B_pad = _round_up(B, Bt)

    time_sb = jnp.pad(time_seq.T, ((0, 0), (0, B_pad - B)))[:, None, :]
    ev_sb = jnp.pad(ev_seq.T, ((0, 0), (0, B_pad - B)))[:, None, :]

    M2 = _round_up(E + 2, _T0)
    embT2 = jnp.zeros((M2, C), mm)
    embT2 = embT2.at[:E].set(embedding.astype(mm).T)
    embT2 = embT2.at[E].set(jnp.ones((C,), mm))
    embT2 = embT2.at[E + 1].set(jnp.ones((C,), mm))

    wx = w_ih.T
    bvec = b_gate[0].astype(f32)
    b_hi = bvec.astype(mm)
    b_lo = (bvec - b_hi.astype(f32)).astype(mm)
    w_cat = jnp.zeros((Gp, _T0 + M2 + H), mm)
    w_cat = w_cat.at[:, 0].set(wx[:, 0])
    w_cat = w_cat.at[:, _T0:_T0 + E].set(wx[:, 1:1 + E])
    w_cat = w_cat.at[:, _T0 + E].set(b_hi)
    w_cat = w_cat.at[:, _T0 + E + 1].set(b_lo)
    w_cat = w_cat.at[:, _T0 + M2:].set(w_hh.T)
    gate_scale = jnp.where(jnp.arange(Gp) < 3 * H, 0.5, 1.0).astype(mm)
    w_cat = w_cat * gate_scale[:, None]
    K = _T0 + M2 + H

    kfn = functools.partial(_rmtpp_fused_kernel, hidden=H, num_classes=C)
    const2 = lambda i: (0, 0)
    out = pl.pallas_call(
        kfn,
        out_shape=jax.ShapeDtypeStruct((B_pad, HEAD), jnp.float32),
        grid=(B_pad // Bt,),
        in_specs=[
            pl.BlockSpec((S, 1, Bt), lambda i: (0, 0, i)),
            pl.BlockSpec((S, 1, Bt), lambda i: (0, 0, i)),
            pl.BlockSpec((M2, C), const2),
            pl.BlockSpec((Gp, K), const2),
            pl.BlockSpec((H, Mp), const2),
            pl.BlockSpec((1, Mp), const2),
            pl.BlockSpec((Mp, HEAD), const2),
            pl.BlockSpec((1, HEAD), const2),
        ],
        out_specs=pl.BlockSpec((Bt, HEAD), lambda i: (i, 0)),
        compiler_params=pltpu.CompilerParams(
            dimension_semantics=("parallel",),
            vmem_limit_bytes=48 << 20),
    )(time_sb, ev_sb, embT2, w_cat, w_map, b_map, w_head, b_head)

    event_out = out[:B, :C]
    time_out = out[:B, C:C + 1]
    last_time = inp[:, -1, 0:1]
    return time_out, event_out, last_time


# speedup vs baseline: 5.5528x; 1.1046x over previous
import functools

import jax
import jax.numpy as jnp
from jax.experimental import pallas as pl
from jax.experimental.pallas import tpu as pltpu


def _round_up(n, m):
    return ((n + m - 1) // m) * m

_T0 = 16


def _rmtpp_fused_kernel(time_ref, ev_ref, embT2_ref, w_cat_ref,
                        w_map_ref, b_map_ref, w_head_ref, b_head_ref,
                        out_ref, *, hidden, num_classes):
    f32 = jnp.float32
    S, _, Bt = time_ref.shape
    H = hidden
    C = num_classes

    embT2 = embT2_ref[...]
    w_cat = w_cat_ref[...]
    mm = w_cat.dtype

    NC = 2
    Bh = Bt // NC
    iota_c = jax.lax.broadcasted_iota(jnp.int32, (C, Bh), 0)
    iota_t = jax.lax.broadcasted_iota(jnp.int32, (_T0, Bh), 0)

    def step(ev, tm, h, c):
        onehot = (iota_c == ev).astype(mm)
        x2 = jnp.dot(embT2, onehot, preferred_element_type=f32).astype(mm)
        tmrow = jnp.where(iota_t == 0, tm.astype(f32), 0.0).astype(mm)
        xh = jnp.concatenate([tmrow, x2, h.astype(mm)], axis=0)
        g = jnp.dot(w_cat, xh, preferred_element_type=f32)
        t3 = jnp.tanh(g[:3 * H])
        i_g = 0.5 * t3[0 * H:1 * H] + 0.5
        f_g = 0.5 * t3[1 * H:2 * H] + 0.5
        o_g = 0.5 * t3[2 * H:3 * H] + 0.5
        g_g = jnp.tanh(g[3 * H:4 * H])
        c = f_g * c + i_g * g_g
        h = o_g * jnp.tanh(c)
        return h, c

    hs = [jnp.zeros((H, Bh), f32)] * NC
    cs = [jnp.zeros((H, Bh), f32)] * NC
    for t in range(S):
        ev_t = ev_ref[t]
        tm_t = time_ref[t]
        for k in range(NC):
            hs[k], cs[k] = step(ev_t[:, k * Bh:(k + 1) * Bh],
                                tm_t[:, k * Bh:(k + 1) * Bh],
                                hs[k], cs[k])

    hT = jnp.concatenate(hs, axis=1)
    h = hT.T.astype(mm)
    map_out = jax.nn.sigmoid(
        jnp.dot(h, w_map_ref[...], preferred_element_type=f32)
        + b_map_ref[...])
    head = (jnp.dot(map_out.astype(mm), w_head_ref[...],
                    preferred_element_type=f32)
            + b_head_ref[...])

    col = jax.lax.broadcasted_iota(jnp.int32, head.shape, 1)
    evt_mask = col < C
    masked = jnp.where(evt_mask, head, jnp.float32(-1e30))
    m = jnp.max(masked, axis=1, keepdims=True)
    p = jnp.where(evt_mask, jnp.exp(masked - m), 0.0)
    lse = m + jnp.log(jnp.sum(p, axis=1, keepdims=True))
    out_ref[...] = jnp.where(evt_mask, head - lse, head)


def kernel(inp, embedding, w_ih, w_hh, b_gate, w_map, b_map, w_head, b_head):
    f32 = jnp.float32
    C, E = embedding.shape
    H = w_hh.shape[0]
    Dp, Gp = w_ih.shape
    Mp = w_map.shape[1]
    HEAD = w_head.shape[1]
    mm = w_ih.dtype
    B, S, _ = inp.shape

    time_seq = inp[:, :, 0]
    ev_seq = inp[:, :, 1].astype(jnp.int32)

    Bt = 4096
    B_pad = _round_up(B, Bt)

    time_sb = jnp.pad(time_seq.T, ((0, 0), (0, B_pad - B))).astype(mm)[:, None, :]
    ev_sb = jnp.pad(ev_seq.T, ((0, 0), (0, B_pad - B)))[:, None, :]

    M2 = _round_up(E + 2, _T0)
    embT2 = jnp.zeros((M2, C), mm)
    embT2 = embT2.at[:E].set(embedding.astype(mm).T)
    embT2 = embT2.at[E].set(jnp.ones((C,), mm))
    embT2 = embT2.at[E + 1].set(jnp.ones((C,), mm))

    wx = w_ih.T
    bvec = b_gate[0].astype(f32)
    b_hi = bvec.astype(mm)
    b_lo = (bvec - b_hi.astype(f32)).astype(mm)
    w_cat = jnp.zeros((Gp, _T0 + M2 + H), mm)
    w_cat = w_cat.at[:, 0].set(wx[:, 0])
    w_cat = w_cat.at[:, _T0:_T0 + E].set(wx[:, 1:1 + E])
    w_cat = w_cat.at[:, _T0 + E].set(b_hi)
    w_cat = w_cat.at[:, _T0 + E + 1].set(b_lo)
    w_cat = w_cat.at[:, _T0 + M2:].set(w_hh.T)
    gate_scale = jnp.where(jnp.arange(Gp) < 3 * H, 0.5, 1.0).astype(mm)
    w_cat = w_cat * gate_scale[:, None]
    K = _T0 + M2 + H

    kfn = functools.partial(_rmtpp_fused_kernel, hidden=H, num_classes=C)
    const2 = lambda i: (0, 0)
    out = pl.pallas_call(
        kfn,
        out_shape=jax.ShapeDtypeStruct((B_pad, HEAD), jnp.float32),
        grid=(B_pad // Bt,),
        in_specs=[
            pl.BlockSpec((S, 1, Bt), lambda i: (0, 0, i)),
            pl.BlockSpec((S, 1, Bt), lambda i: (0, 0, i)),
            pl.BlockSpec((M2, C), const2),
            pl.BlockSpec((Gp, K), const2),
            pl.BlockSpec((H, Mp), const2),
            pl.BlockSpec((1, Mp), const2),
            pl.BlockSpec((Mp, HEAD), const2),
            pl.BlockSpec((1, HEAD), const2),
        ],
        out_specs=pl.BlockSpec((Bt, HEAD), lambda i: (i, 0)),
        compiler_params=pltpu.CompilerParams(
            dimension_semantics=("parallel",),
            vmem_limit_bytes=56 << 20),
    )(time_sb, ev_sb, embT2, w_cat, w_map, b_map, w_head, b_head)

    event_out = out[:B, :C]
    time_out = out[:B, C:C + 1]
    last_time = inp[:, -1, 0:1]
    return time_out, event_out, last_time


# speedup vs baseline: 5.6542x; 1.0183x over previous
import functools

import jax
import jax.numpy as jnp
from jax.experimental import pallas as pl
from jax.experimental.pallas import tpu as pltpu


def _round_up(n, m):
    return ((n + m - 1) // m) * m

_T0 = 16


def _rmtpp_fused_kernel(time_ref, ev_ref, embT2_ref, w_cat_ref,
                        w_map_ref, b_map_ref, w_head_ref, b_head_ref,
                        out_ref, *, hidden, num_classes):
    f32 = jnp.float32
    S, _, Bt = time_ref.shape
    H = hidden
    C = num_classes

    embT2 = embT2_ref[...]
    w_cat = w_cat_ref[...]
    mm = w_cat.dtype

    NC = 2
    Bh = Bt // NC
    iota_c = jax.lax.broadcasted_iota(jnp.int32, (C, Bh), 0)
    iota_t = jax.lax.broadcasted_iota(jnp.int32, (_T0, Bh), 0)

    def step(ev, tm, h, c):
        onehot = (iota_c == ev).astype(mm)
        x2 = jnp.dot(embT2, onehot, preferred_element_type=f32).astype(mm)
        tmrow = jnp.where(iota_t == 0, tm.astype(f32), 0.0).astype(mm)
        xh = jnp.concatenate([tmrow, x2, h], axis=0)
        g = jnp.dot(w_cat, xh, preferred_element_type=f32)
        t3 = jnp.tanh(g[:3 * H])
        i_t = t3[0 * H:1 * H]
        f_t = t3[1 * H:2 * H]
        o_t = t3[2 * H:3 * H]
        g_g = jnp.tanh(g[3 * H:4 * H])
        c = 0.5 * ((c + f_t * c) + (g_g + i_t * g_g))
        tc = jnp.tanh(c)
        h = (0.5 * (tc + o_t * tc)).astype(mm)
        return h, c

    hs = [jnp.zeros((H, Bh), mm)] * NC
    cs = [jnp.zeros((H, Bh), f32)] * NC
    for t in range(S):
        ev_t = ev_ref[t]
        tm_t = time_ref[t]
        for k in range(NC):
            hs[k], cs[k] = step(ev_t[:, k * Bh:(k + 1) * Bh],
                                tm_t[:, k * Bh:(k + 1) * Bh],
                                hs[k], cs[k])

    hT = jnp.concatenate(hs, axis=1)
    h = hT.T.astype(mm)
    map_out = jax.nn.sigmoid(
        jnp.dot(h, w_map_ref[...], preferred_element_type=f32)
        + b_map_ref[...])
    head = (jnp.dot(map_out.astype(mm), w_head_ref[...],
                    preferred_element_type=f32)
            + b_head_ref[...])

    col = jax.lax.broadcasted_iota(jnp.int32, head.shape, 1)
    evt_mask = col < C
    masked = jnp.where(evt_mask, head, jnp.float32(-1e30))
    m = jnp.max(masked, axis=1, keepdims=True)
    p = jnp.where(evt_mask, jnp.exp(masked - m), 0.0)
    lse = m + jnp.log(jnp.sum(p, axis=1, keepdims=True))
    out_ref[...] = jnp.where(evt_mask, head - lse, head)


def kernel(inp, embedding, w_ih, w_hh, b_gate, w_map, b_map, w_head, b_head):
    f32 = jnp.float32
    C, E = embedding.shape
    H = w_hh.shape[0]
    Dp, Gp = w_ih.shape
    Mp = w_map.shape[1]
    HEAD = w_head.shape[1]
    mm = w_ih.dtype
    B, S, _ = inp.shape

    time_seq = inp[:, :, 0]
    ev_seq = inp[:, :, 1].astype(jnp.int32)

    Bt = 4096
    B_pad = _round_up(B, Bt)

    time_sb = jnp.pad(time_seq.T, ((0, 0), (0, B_pad - B))).astype(mm)[:, None, :]
    ev_sb = jnp.pad(ev_seq.T, ((0, 0), (0, B_pad - B)))[:, None, :]

    M2 = _round_up(E + 2, _T0)
    embT2 = jnp.zeros((M2, C), mm)
    embT2 = embT2.at[:E].set(embedding.astype(mm).T)
    embT2 = embT2.at[E].set(jnp.ones((C,), mm))
    embT2 = embT2.at[E + 1].set(jnp.ones((C,), mm))

    wx = w_ih.T
    bvec = b_gate[0].astype(f32)
    b_hi = bvec.astype(mm)
    b_lo = (bvec - b_hi.astype(f32)).astype(mm)
    w_cat = jnp.zeros((Gp, _T0 + M2 + H), mm)
    w_cat = w_cat.at[:, 0].set(wx[:, 0])
    w_cat = w_cat.at[:, _T0:_T0 + E].set(wx[:, 1:1 + E])
    w_cat = w_cat.at[:, _T0 + E].set(b_hi)
    w_cat = w_cat.at[:, _T0 + E + 1].set(b_lo)
    w_cat = w_cat.at[:, _T0 + M2:].set(w_hh.T)
    gate_scale = jnp.where(jnp.arange(Gp) < 3 * H, 0.5, 1.0).astype(mm)
    w_cat = w_cat * gate_scale[:, None]
    K = _T0 + M2 + H

    kfn = functools.partial(_rmtpp_fused_kernel, hidden=H, num_classes=C)
    const2 = lambda i: (0, 0)
    out = pl.pallas_call(
        kfn,
        out_shape=jax.ShapeDtypeStruct((B_pad, HEAD), jnp.float32),
        grid=(B_pad // Bt,),
        in_specs=[
            pl.BlockSpec((S, 1, Bt), lambda i: (0, 0, i)),
            pl.BlockSpec((S, 1, Bt), lambda i: (0, 0, i)),
            pl.BlockSpec((M2, C), const2),
            pl.BlockSpec((Gp, K), const2),
            pl.BlockSpec((H, Mp), const2),
            pl.BlockSpec((1, Mp), const2),
            pl.BlockSpec((Mp, HEAD), const2),
            pl.BlockSpec((1, HEAD), const2),
        ],
        out_specs=pl.BlockSpec((Bt, HEAD), lambda i: (i, 0)),
        compiler_params=pltpu.CompilerParams(
            dimension_semantics=("parallel",),
            vmem_limit_bytes=56 << 20),
    )(time_sb, ev_sb, embT2, w_cat, w_map, b_map, w_head, b_head)

    event_out = out[:B, :C]
    time_out = out[:B, C:C + 1]
    last_time = inp[:, -1, 0:1]
    return time_out, event_out, last_time
